# baseline (speedup 1.0000x reference)
import functools

import jax
import jax.numpy as jnp
from jax import lax
from jax.experimental import pallas as pl
from jax.experimental.pallas import tpu as pltpu
from jax.experimental.pallas import tpu_sc as plsc

H = 4096
W = 4096
C = 3
N_SAMPLES = 1_000_000

NC = 2
NS = 16
NW = NC * NS
LANES = 16

B = 1024
CHUNKS = 31
NPW = B * CHUNKS
NPAD = NPW * NW


@functools.partial(
    pl.kernel,
    mesh=plsc.VectorSubcoreMesh(core_axis_name="c", subcore_axis_name="s"),
    compiler_params=pltpu.CompilerParams(
        needs_layout_passes=False, use_tc_tiling_on_sc=False),
    out_type=jax.ShapeDtypeStruct((C * NPAD,), jnp.float32),
    scratch_types=[
        pltpu.VMEM((2 * B,), jnp.float32),
        [pltpu.VMEM((B,), jnp.int32) for _ in range(12)],
        [pltpu.VMEM((B,), jnp.float32) for _ in range(4)],
        [pltpu.VMEM((B,), jnp.float32) for _ in range(12)],
        [pltpu.VMEM((B,), jnp.float32) for _ in range(3)],
        pltpu.SemaphoreType.DMA,
    ],
)
def _bilerp(xs_hbm, data_hbm, out_hbm, xs_v, idx_v, w_v, g_v, o_v, sem):
    wid = lax.axis_index("s") * NC + lax.axis_index("c")
    iota = lax.iota(jnp.int32, LANES)

    def chunk_body(k, _):
        base = wid * NPW + k * B
        pltpu.sync_copy(xs_hbm.at[pl.ds(2 * base, 2 * B)], xs_v)

        def ph1(j, _):
            n = j * LANES
            xi = 2 * (n + iota)
            xv = plsc.load_gather(xs_v, [xi])
            yv = plsc.load_gather(xs_v, [xi + 1])
            sx = xv * jnp.float32(W)
            sy = yv * jnp.float32(H)
            ix = sx.astype(jnp.int32)
            iy = sy.astype(jnp.int32)
            fx = sx - ix.astype(jnp.float32)
            fy = sy - iy.astype(jnp.float32)
            x0 = jnp.clip(ix, 0, W - 1)
            y0 = jnp.clip(iy, 0, H - 1)
            x1 = jnp.minimum(x0 + 1, W - 1)
            y1 = jnp.minimum(y0 + 1, H - 1)
            yb0 = y0 * (C * W)
            yb1 = y1 * (C * W)
            xb0 = x0 * C
            xb1 = x1 * C
            e = [yb0 + xb0, yb0 + xb1, yb1 + xb0, yb1 + xb1]
            sl = pl.ds(n, LANES)
            for kk in range(4):
                for cc in range(C):
                    idx_v[kk * C + cc][sl] = e[kk] + cc
            gx = 1.0 - fx
            gy = 1.0 - fy
            w_v[0][sl] = gx * gy
            w_v[1][sl] = fx * gy
            w_v[2][sl] = gx * fy
            w_v[3][sl] = fx * fy
            return 0

        lax.fori_loop(0, B // LANES, ph1, 0, unroll=2)

        copies = [pltpu.async_copy(data_hbm.at[idx_v[q]], g_v[q], sem)
                  for q in range(12)]
        for cp in copies:
            cp.wait()

        def ph2(j, _):
            sl = pl.ds(j * LANES, LANES)
            ws = [w_v[kk][sl] for kk in range(4)]
            for cc in range(C):
                acc = g_v[cc][sl] * ws[0]
                acc = acc + g_v[C + cc][sl] * ws[1]
                acc = acc + g_v[2 * C + cc][sl] * ws[2]
                acc = acc + g_v[3 * C + cc][sl] * ws[3]
                o_v[cc][sl] = acc
            return 0

        lax.fori_loop(0, B // LANES, ph2, 0, unroll=2)

        for cc in range(C):
            pltpu.sync_copy(o_v[cc], out_hbm.at[pl.ds(cc * NPAD + base, B)])
        return 0

    lax.fori_loop(0, CHUNKS, chunk_body, 0)


def kernel(xs, data):
    xs_flat = jnp.ravel(xs)
    xs_flat = jnp.concatenate(
        [xs_flat, jnp.zeros((2 * NPAD - 2 * N_SAMPLES,), jnp.float32)])
    table = jnp.ravel(data)
    out_planar = _bilerp(xs_flat, table)
    return out_planar.reshape(C, NPAD)[:, :N_SAMPLES].T


# speedup vs baseline: 75.8731x; 75.8731x over previous
import functools

import jax
import jax.numpy as jnp
from jax import lax
from jax.experimental import pallas as pl
from jax.experimental.pallas import tpu as pltpu
from jax.experimental.pallas import tpu_sc as plsc

H = 4096
W = 4096
C = 3
PLANE = H * W
N_SAMPLES = 1_000_000

NC = 2
NS = 16
NW = NC * NS
LANES = 16

B = 1024
CHUNKS = 31
NPW = B * CHUNKS
NPAD = NPW * NW


@functools.partial(
    pl.kernel,
    mesh=plsc.VectorSubcoreMesh(core_axis_name="c", subcore_axis_name="s"),
    compiler_params=pltpu.CompilerParams(
        needs_layout_passes=False, use_tc_tiling_on_sc=False),
    out_type=jax.ShapeDtypeStruct((C * NPAD,), jnp.float32),
    scratch_types=[
        pltpu.VMEM((B,), jnp.float32),
        pltpu.VMEM((B,), jnp.float32),
        [pltpu.VMEM((B,), jnp.int32) for _ in range(12)],
        [pltpu.VMEM((B,), jnp.float32) for _ in range(4)],
        [pltpu.VMEM((B,), jnp.float32) for _ in range(12)],
        [pltpu.VMEM((B,), jnp.float32) for _ in range(3)],
        pltpu.SemaphoreType.DMA,
    ],
)
def _bilerp(xsx_hbm, xsy_hbm, data_hbm, out_hbm,
            xx_v, yy_v, idx_v, w_v, g_v, o_v, sem):
    wid = lax.axis_index("s") * NC + lax.axis_index("c")

    def chunk_body(k, _):
        base = wid * NPW + k * B
        pltpu.sync_copy(xsx_hbm.at[pl.ds(base, B)], xx_v)
        pltpu.sync_copy(xsy_hbm.at[pl.ds(base, B)], yy_v)

        def ph1(j, _):
            sl = pl.ds(j * LANES, LANES)
            sx = xx_v[sl] * jnp.float32(W)
            sy = yy_v[sl] * jnp.float32(H)
            ix = sx.astype(jnp.int32)
            iy = sy.astype(jnp.int32)
            fx = sx - ix.astype(jnp.float32)
            fy = sy - iy.astype(jnp.float32)
            x0 = jnp.clip(ix, 0, W - 1)
            y0 = jnp.clip(iy, 0, H - 1)
            x1 = jnp.minimum(x0 + 1, W - 1)
            y1 = jnp.minimum(y0 + 1, H - 1)
            yb0 = y0 << 12
            yb1 = y1 << 12
            e = [yb0 + x0, yb0 + x1, yb1 + x0, yb1 + x1]
            for kk in range(4):
                for cc in range(C):
                    idx_v[kk * C + cc][sl] = e[kk] + (cc * PLANE)
            gx = 1.0 - fx
            gy = 1.0 - fy
            w_v[0][sl] = gx * gy
            w_v[1][sl] = fx * gy
            w_v[2][sl] = gx * fy
            w_v[3][sl] = fx * fy
            return 0

        lax.fori_loop(0, B // LANES, ph1, 0, unroll=2)

        copies = [pltpu.async_copy(data_hbm.at[idx_v[q]], g_v[q], sem)
                  for q in range(12)]
        for cp in copies:
            cp.wait()

        def ph2(j, _):
            sl = pl.ds(j * LANES, LANES)
            ws = [w_v[kk][sl] for kk in range(4)]
            for cc in range(C):
                acc = g_v[cc][sl] * ws[0]
                acc = acc + g_v[C + cc][sl] * ws[1]
                acc = acc + g_v[2 * C + cc][sl] * ws[2]
                acc = acc + g_v[3 * C + cc][sl] * ws[3]
                o_v[cc][sl] = acc
            return 0

        lax.fori_loop(0, B // LANES, ph2, 0, unroll=2)

        for cc in range(C):
            pltpu.sync_copy(o_v[cc], out_hbm.at[pl.ds(cc * NPAD + base, B)])
        return 0

    lax.fori_loop(0, CHUNKS, chunk_body, 0)


def kernel(xs, data):
    npad = NPAD - N_SAMPLES
    spread = (jnp.arange(npad, dtype=jnp.float32) % 4093.0) / 4096.0
    xs_x = jnp.concatenate([xs[:, 0], spread])
    xs_y = jnp.concatenate([xs[:, 1], spread])
    table = data.transpose(2, 0, 1).reshape(49152, 8, 128).reshape(C * PLANE)
    out_planar = _bilerp(xs_x, xs_y, table)
    return out_planar.reshape(C, NPAD)[:, :N_SAMPLES].T


# speedup vs baseline: 85.8169x; 1.1311x over previous
import functools

import jax
import jax.numpy as jnp
from jax import lax
from jax.experimental import pallas as pl
from jax.experimental.pallas import tpu as pltpu
from jax.experimental.pallas import tpu_sc as plsc

H = 4096
W = 4096
C = 3
PLANE = H * W
N_SAMPLES = 1_000_000

NC = 2
NS = 16
NW = NC * NS
LANES = 16

B = 992
CHUNKS = 32
NPW = B * CHUNKS
NPAD = NPW * NW

_SET = lambda: [
    pltpu.VMEM((B,), jnp.float32),
    pltpu.VMEM((B,), jnp.float32),
    [pltpu.VMEM((B,), jnp.int32) for _ in range(12)],
    [pltpu.VMEM((B,), jnp.float32) for _ in range(4)],
    [pltpu.VMEM((B,), jnp.float32) for _ in range(12)],
]


@functools.partial(
    pl.kernel,
    mesh=plsc.VectorSubcoreMesh(core_axis_name="c", subcore_axis_name="s"),
    compiler_params=pltpu.CompilerParams(
        needs_layout_passes=False, use_tc_tiling_on_sc=False),
    out_type=jax.ShapeDtypeStruct((C * NPAD,), jnp.float32),
    scratch_types=[
        _SET(), _SET(),
        [pltpu.VMEM((B,), jnp.float32) for _ in range(3)],
        pltpu.SemaphoreType.DMA,
        pltpu.SemaphoreType.DMA,
    ],
)
def _bilerp(xsx_hbm, xsy_hbm, data_hbm, out_hbm,
            set0, set1, o_v, sem0, sem1):
    wid = lax.axis_index("s") * NC + lax.axis_index("c")
    wbase = wid * NPW
    sets = (set0, set1)
    sems = (sem0, sem1)

    def ph1(k, s):
        xx_v, yy_v, idx_v, w_v, _ = sets[s]
        base = wbase + k * B
        pltpu.sync_copy(xsx_hbm.at[pl.ds(base, B)], xx_v)
        pltpu.sync_copy(xsy_hbm.at[pl.ds(base, B)], yy_v)

        def body(j, _):
            sl = pl.ds(j * LANES, LANES)
            sx = xx_v[sl] * jnp.float32(W)
            sy = yy_v[sl] * jnp.float32(H)
            ix = sx.astype(jnp.int32)
            iy = sy.astype(jnp.int32)
            fx = sx - ix.astype(jnp.float32)
            fy = sy - iy.astype(jnp.float32)
            x0 = jnp.clip(ix, 0, W - 1)
            y0 = jnp.clip(iy, 0, H - 1)
            x1 = jnp.minimum(x0 + 1, W - 1)
            y1 = jnp.minimum(y0 + 1, H - 1)
            yb0 = y0 << 12
            yb1 = y1 << 12
            e = [yb0 + x0, yb0 + x1, yb1 + x0, yb1 + x1]
            for kk in range(4):
                for cc in range(C):
                    idx_v[kk * C + cc][sl] = e[kk] + (cc * PLANE)
            gx = 1.0 - fx
            gy = 1.0 - fy
            w_v[0][sl] = gx * gy
            w_v[1][sl] = fx * gy
            w_v[2][sl] = gx * fy
            w_v[3][sl] = fx * fy
            return 0

        lax.fori_loop(0, B // LANES, body, 0, unroll=2)

    def fire(s):
        _, _, idx_v, _, g_v = sets[s]
        for q in range(12):
            pltpu.async_copy(data_hbm.at[idx_v[q]], g_v[q], sems[s])

    def wait(s):
        _, _, idx_v, _, g_v = sets[s]
        for q in range(12):
            pltpu.make_async_copy(data_hbm.at[idx_v[q]], g_v[q], sems[s]).wait()

    def ph3(k, s):
        _, _, _, w_v, g_v = sets[s]
        base = wbase + k * B

        def body(j, _):
            sl = pl.ds(j * LANES, LANES)
            ws = [w_v[kk][sl] for kk in range(4)]
            for cc in range(C):
                acc = g_v[cc][sl] * ws[0]
                acc = acc + g_v[C + cc][sl] * ws[1]
                acc = acc + g_v[2 * C + cc][sl] * ws[2]
                acc = acc + g_v[3 * C + cc][sl] * ws[3]
                o_v[cc][sl] = acc
            return 0

        lax.fori_loop(0, B // LANES, body, 0, unroll=2)
        for cc in range(C):
            pltpu.sync_copy(o_v[cc], out_hbm.at[pl.ds(cc * NPAD + base, B)])

    ph1(0, 0)
    fire(0)

    def pair(i, _):
        k0 = 2 * i
        k1 = k0 + 1
        ph1(k1, 1)
        fire(1)
        wait(0)
        ph3(k0, 0)

        @pl.when(i < (CHUNKS // 2) - 1)
        def _():
            ph1(k0 + 2, 0)
            fire(0)

        wait(1)
        ph3(k1, 1)
        return 0

    lax.fori_loop(0, CHUNKS // 2, pair, 0)


def kernel(xs, data):
    npad = NPAD - N_SAMPLES
    spread = (jnp.arange(npad, dtype=jnp.float32) % 4093.0) / 4096.0
    xs_x = jnp.concatenate([xs[:, 0], spread])
    xs_y = jnp.concatenate([xs[:, 1], spread])
    table = data.transpose(2, 0, 1).reshape(49152, 8, 128).reshape(C * PLANE)
    out_planar = _bilerp(xs_x, xs_y, table)
    return out_planar.reshape(C, NPAD)[:, :N_SAMPLES].T
